# Initial kernel scaffold; baseline (speedup 1.0000x reference)
import jax
import jax.numpy as jnp
from jax.experimental import pallas as pl


def kernel(x, embeddings):
    raise NotImplementedError("write your pallas kernel here")


# speedup vs baseline: 1.1080x; 1.1080x over previous
import functools

import jax
import jax.numpy as jnp
from jax import lax
from jax.experimental import pallas as pl
from jax.experimental.pallas import tpu as pltpu
from jax.experimental.pallas import tpu_sc as plsc

VOCAB = 1000000
DIM = 32
BATCH = 16384
HIST = 50
TOTAL = BATCH * HIST

_info = plsc.get_sparse_core_info()
_NC = _info.num_cores
_NS = _info.num_subcores
_NW = _NC * _NS
_B_PER_W = TOTAL // _NW
_CHUNK = 1600
_NCHUNK = _B_PER_W // _CHUNK


def _make_gather():
    mesh = plsc.VectorSubcoreMesh(core_axis_name="c", subcore_axis_name="s")

    @functools.partial(
        pl.kernel,
        mesh=mesh,
        out_type=jax.ShapeDtypeStruct((TOTAL, DIM), jnp.float32),
        scratch_types=[
            pltpu.VMEM((_B_PER_W,), jnp.int32),
            pltpu.VMEM((_CHUNK, DIM), jnp.float32),
            pltpu.SemaphoreType.DMA,
        ],
        compiler_params=pltpu.CompilerParams(use_tc_tiling_on_sc=False),
    )
    def gather_kernel(idx_hbm, table_hbm, out_hbm, idx_v, rows_v, sem):
        wid = lax.axis_index("s") * _NC + lax.axis_index("c")
        base = wid * _B_PER_W
        pltpu.sync_copy(idx_hbm.at[pl.ds(base, _B_PER_W)], idx_v)
        for i in range(_NCHUNK):
            pltpu.async_copy(
                table_hbm.at[idx_v.at[pl.ds(i * _CHUNK, _CHUNK)]],
                rows_v,
                sem,
            ).wait()
            pltpu.sync_copy(rows_v, out_hbm.at[pl.ds(base + i * _CHUNK, _CHUNK)])

    return gather_kernel


_gather = _make_gather()


@jax.jit
def kernel(x, embeddings):
    idx = x.reshape(TOTAL).astype(jnp.int32)
    rows = _gather(idx, embeddings)
    return rows.reshape(BATCH, HIST, DIM)


# speedup vs baseline: 1.1120x; 1.0036x over previous
import functools

import jax
import jax.numpy as jnp
from jax import lax
from jax.experimental import pallas as pl
from jax.experimental.pallas import tpu as pltpu
from jax.experimental.pallas import tpu_sc as plsc

VOCAB = 1000000
DIM = 32
BATCH = 16384
HIST = 50
TOTAL = BATCH * HIST

_info = plsc.get_sparse_core_info()
_NC = _info.num_cores
_NS = _info.num_subcores
_NW = _NC * _NS
_B_PER_W = TOTAL // _NW
_CHUNK = 1024
_NBUF = 3
_NCHUNK = _B_PER_W // _CHUNK


def _make_gather():
    mesh = plsc.VectorSubcoreMesh(core_axis_name="c", subcore_axis_name="s")

    @functools.partial(
        pl.kernel,
        mesh=mesh,
        out_type=jax.ShapeDtypeStruct((TOTAL, DIM), jnp.float32),
        scratch_types=[
            pltpu.VMEM((_B_PER_W,), jnp.int32),
            [pltpu.VMEM((_CHUNK, DIM), jnp.float32) for _ in range(_NBUF)],
            [pltpu.SemaphoreType.DMA for _ in range(_NBUF)],
            [pltpu.SemaphoreType.DMA for _ in range(_NBUF)],
        ],
        compiler_params=pltpu.CompilerParams(use_tc_tiling_on_sc=False),
    )
    def gather_kernel(idx_hbm, table_hbm, out_hbm, idx_v, bufs, gsems, ssems):
        wid = lax.axis_index("s") * _NC + lax.axis_index("c")
        base = wid * _B_PER_W
        pltpu.sync_copy(idx_hbm.at[pl.ds(base, _B_PER_W)], idx_v)

        def start_gather(i):
            b = i % _NBUF
            return pltpu.async_copy(
                table_hbm.at[idx_v.at[pl.ds(i * _CHUNK, _CHUNK)]],
                bufs[b],
                gsems[b],
            )

        def start_store(i):
            b = i % _NBUF
            return pltpu.async_copy(
                bufs[b],
                out_hbm.at[pl.ds(base + i * _CHUNK, _CHUNK)],
                ssems[b],
            )

        gathers = [None] * _NCHUNK
        stores = [None] * _NCHUNK
        for i in range(min(_NBUF, _NCHUNK)):
            gathers[i] = start_gather(i)
        for i in range(_NCHUNK):
            nxt = i + 1
            if _NBUF <= nxt < _NCHUNK:
                stores[nxt - _NBUF].wait()
                gathers[nxt] = start_gather(nxt)
            gathers[i].wait()
            stores[i] = start_store(i)
        for i in range(max(0, _NCHUNK - _NBUF), _NCHUNK):
            stores[i].wait()

    return gather_kernel


_gather = _make_gather()


@jax.jit
def kernel(x, embeddings):
    idx = x.reshape(TOTAL).astype(jnp.int32)
    rows = _gather(idx, embeddings)
    return rows.reshape(BATCH, HIST, DIM)


# speedup vs baseline: 1.1202x; 1.0074x over previous
import functools

import jax
import jax.numpy as jnp
from jax import lax
from jax.experimental import pallas as pl
from jax.experimental.pallas import tpu as pltpu
from jax.experimental.pallas import tpu_sc as plsc

VOCAB = 1000000
DIM = 32
BATCH = 16384
HIST = 50
TOTAL = BATCH * HIST

_info = plsc.get_sparse_core_info()
_NC = _info.num_cores
_NS = _info.num_subcores
_NW = _NC * _NS
_B_PER_W = TOTAL // _NW
_CHUNK = 1024
_NBUF = 3
_NCHUNK = _B_PER_W // _CHUNK


def _make_gather():
    mesh = plsc.VectorSubcoreMesh(core_axis_name="c", subcore_axis_name="s")

    @functools.partial(
        pl.kernel,
        mesh=mesh,
        out_type=jax.ShapeDtypeStruct((TOTAL, DIM), jnp.float32),
        scratch_types=[
            pltpu.VMEM((_B_PER_W,), jnp.int32),
            [pltpu.VMEM((_CHUNK, DIM), jnp.float32) for _ in range(_NBUF)],
            [pltpu.SemaphoreType.DMA for _ in range(_NBUF)],
            [pltpu.SemaphoreType.DMA for _ in range(_NBUF)],
        ],
        compiler_params=pltpu.CompilerParams(use_tc_tiling_on_sc=False),
    )
    def gather_kernel(idx_hbm, table_hbm, out_hbm, idx_v, bufs, gsems, ssems):
        wid = lax.axis_index("s") * _NC + lax.axis_index("c")
        base = wid * _B_PER_W
        pltpu.sync_copy(idx_hbm.at[pl.ds(base, _B_PER_W)], idx_v)

        def start_gather(i):
            b = i % _NBUF
            return pltpu.async_copy(
                table_hbm.at[idx_v.at[pl.ds(i * _CHUNK, _CHUNK)]],
                bufs[b],
                gsems[b],
            )

        def start_store(i):
            b = i % _NBUF
            return pltpu.async_copy(
                bufs[b],
                out_hbm.at[pl.ds(base + i * _CHUNK, _CHUNK)],
                ssems[b],
            )

        gathers = [None] * _NCHUNK
        stores = [None] * _NCHUNK
        for i in range(min(_NBUF, _NCHUNK)):
            gathers[i] = start_gather(i)
        for i in range(_NCHUNK):
            nxt = i + 1
            if _NBUF <= nxt < _NCHUNK:
                stores[nxt - _NBUF].wait()
                gathers[nxt] = start_gather(nxt)
            gathers[i].wait()
            stores[i] = start_store(i)
        for i in range(max(0, _NCHUNK - _NBUF), _NCHUNK):
            stores[i].wait()

    return gather_kernel


_gather = _make_gather()


@jax.jit
def kernel(x, embeddings):
    tpad = jnp.pad(embeddings, ((0, 0), (0, 128 - DIM))).reshape(4 * VOCAB, DIM)
    idx = x.reshape(TOTAL).astype(jnp.int32) * 4
    rows = _gather(idx, tpad)
    return rows.reshape(BATCH, HIST, DIM)


# speedup vs baseline: 1.6066x; 1.4341x over previous
import functools

import jax
import jax.numpy as jnp
from jax import lax
from jax.experimental import pallas as pl
from jax.experimental.pallas import tpu as pltpu
from jax.experimental.pallas import tpu_sc as plsc

VOCAB = 1000000
DIM = 32
BATCH = 16384
HIST = 50
TOTAL = BATCH * HIST

_info = plsc.get_sparse_core_info()
_NC = _info.num_cores
_NS = _info.num_subcores
_NW = _NC * _NS
_BW = BATCH // _NW
_BT = _BW // 128


def _make_kernel():
    mesh = plsc.VectorSubcoreMesh(core_axis_name="c", subcore_axis_name="s")

    @functools.partial(
        pl.kernel,
        mesh=mesh,
        out_type=jax.ShapeDtypeStruct((HIST, DIM // 8, BATCH // 128, 8, 128),
                                      jnp.float32),
        scratch_types=[
            pltpu.VMEM((HIST * _BW,), jnp.int32),
            [pltpu.VMEM((_BW, DIM), jnp.float32) for _ in range(2)],
            [pltpu.VMEM((DIM // 8, _BT, 8, 128), jnp.float32) for _ in range(2)],
            pltpu.SemaphoreType.DMA,
            [pltpu.SemaphoreType.DMA for _ in range(2)],
        ],
        compiler_params=pltpu.CompilerParams(
            use_tc_tiling_on_sc=False, needs_layout_passes=False
        ),
    )
    def k(idx_hbm, table_hbm, out_hbm, idx_all, rows, obuf, isem, gsems):
        w = lax.axis_index("s") * _NC + lax.axis_index("c")
        b0 = w * _BW

        idx_copies = []
        for h in range(HIST):
            idx_copies.append(pltpu.async_copy(
                idx_hbm.at[pl.ds(h * BATCH + b0, _BW)],
                idx_all.at[pl.ds(h * _BW, _BW)],
                isem,
            ))
        for c in idx_copies:
            c.wait()

        def start_gather(h, p):
            return pltpu.async_copy(
                table_hbm.at[idx_all.at[pl.ds(h * _BW, _BW)]],
                rows[p],
                gsems[p],
            )

        bvecs = [lax.iota(jnp.int32, 16) + 16 * j for j in range(_BW // 16)]

        def transpose_unit(p):
            rp, op = rows[p], obuf[p]

            def dbody(d, _):
                dt = d // 8
                di = d % 8
                dvec = jnp.full((16,), d, jnp.int32)
                for j in range(_BW // 16):
                    v = plsc.load_gather(rp, [bvecs[j], dvec])
                    op[dt, j // 8, di, pl.ds((j % 8) * 16, 16)] = v
                return _

            lax.fori_loop(0, DIM, dbody, None)

        g0 = start_gather(0, 0)
        g1 = start_gather(1, 1)
        del g0, g1

        def unit(h, p):
            pltpu.make_async_copy(
                table_hbm.at[idx_all.at[pl.ds(h * _BW, _BW)]],
                rows[p],
                gsems[p],
            ).wait()
            transpose_unit(p)

            @pl.when(h + 2 < HIST)
            def _():
                start_gather(h + 2, p)

            for dt in range(DIM // 8):
                pltpu.sync_copy(
                    obuf[p].at[dt],
                    out_hbm.at[h, dt, pl.ds(_BT * w, _BT)],
                )

        def base_body(base, _):
            unit(2 * base, 0)
            unit(2 * base + 1, 1)
            return _

        lax.fori_loop(0, HIST // 2, base_body, None)

    return k


_k = _make_kernel()


@jax.jit
def kernel(x, embeddings):
    tpad = jnp.pad(embeddings, ((0, 0), (0, 128 - DIM))).reshape(4 * VOCAB, DIM)
    idx = x.T.reshape(TOTAL).astype(jnp.int32) * 4
    y6 = _k(idx, tpad)
    return y6.transpose(2, 4, 0, 1, 3).reshape(BATCH, HIST, DIM)


# speedup vs baseline: 1.6667x; 1.0374x over previous
import functools

import jax
import jax.numpy as jnp
from jax import lax
from jax.experimental import pallas as pl
from jax.experimental.pallas import tpu as pltpu
from jax.experimental.pallas import tpu_sc as plsc

VOCAB = 1000000
DIM = 32
BATCH = 16384
HIST = 50
TOTAL = BATCH * HIST

_info = plsc.get_sparse_core_info()
_NC = _info.num_cores
_NS = _info.num_subcores
_NW = _NC * _NS
_BW = BATCH // _NW
_BT = _BW // 128


def _make_kernel():
    mesh = plsc.VectorSubcoreMesh(core_axis_name="c", subcore_axis_name="s")

    @functools.partial(
        pl.kernel,
        mesh=mesh,
        out_type=jax.ShapeDtypeStruct((HIST, DIM // 8, BATCH // 128, 8, 128),
                                      jnp.float32),
        scratch_types=[
            pltpu.VMEM((HIST * _BW,), jnp.int32),
            [pltpu.VMEM((_BW, DIM), jnp.float32) for _ in range(2)],
            [pltpu.VMEM((DIM // 8, _BT, 8, 128), jnp.float32) for _ in range(2)],
            pltpu.SemaphoreType.DMA,
            [pltpu.SemaphoreType.DMA for _ in range(2)],
            [pltpu.SemaphoreType.DMA for _ in range(2)],
        ],
        compiler_params=pltpu.CompilerParams(
            use_tc_tiling_on_sc=False, needs_layout_passes=False
        ),
    )
    def k(idx_hbm, table_hbm, out_hbm, idx_all, rows, obuf, isem, gsems, ssems):
        w = lax.axis_index("s") * _NC + lax.axis_index("c")
        b0 = w * _BW

        idx_copies = []
        for h in range(HIST):
            idx_copies.append(pltpu.async_copy(
                idx_hbm.at[pl.ds(h * BATCH + b0, _BW)],
                idx_all.at[pl.ds(h * _BW, _BW)],
                isem,
            ))
        for c in idx_copies:
            c.wait()

        def start_gather(h, p):
            return pltpu.async_copy(
                table_hbm.at[idx_all.at[pl.ds(h * _BW, _BW)]],
                rows[p],
                gsems[p],
            )

        bvecs = [lax.iota(jnp.int32, 16) + 16 * j for j in range(_BW // 16)]

        def transpose_unit(p):
            rp, op = rows[p], obuf[p]

            def dbody(d, _):
                dt = d // 8
                di = d % 8
                dvec = jnp.full((16,), d, jnp.int32)
                for j in range(_BW // 16):
                    v = plsc.load_gather(rp, [bvecs[j], dvec])
                    op[dt, j // 8, di, pl.ds((j % 8) * 16, 16)] = v
                return _

            lax.fori_loop(0, DIM, dbody, None)

        g0 = start_gather(0, 0)
        g1 = start_gather(1, 1)
        del g0, g1

        def store_ref(h):
            return out_hbm.at[h, :, pl.ds(_BT * w, _BT)]

        def unit(h, p):
            pltpu.make_async_copy(
                table_hbm.at[idx_all.at[pl.ds(h * _BW, _BW)]],
                rows[p],
                gsems[p],
            ).wait()

            @pl.when(h >= 2)
            def _():
                pltpu.make_async_copy(obuf[p], store_ref(h), ssems[p]).wait()

            transpose_unit(p)

            @pl.when(h + 2 < HIST)
            def _():
                start_gather(h + 2, p)

            pltpu.async_copy(obuf[p], store_ref(h), ssems[p])

        def base_body(base, _):
            unit(2 * base, 0)
            unit(2 * base + 1, 1)
            return _

        lax.fori_loop(0, HIST // 2, base_body, None)
        for p in range(2):
            pltpu.make_async_copy(obuf[p], store_ref(HIST - 2 + p), ssems[p]).wait()

    return k


_k = _make_kernel()


@jax.jit
def kernel(x, embeddings):
    tpad = jnp.pad(embeddings, ((0, 0), (0, 128 - DIM))).reshape(4 * VOCAB, DIM)
    idx = x.T.reshape(TOTAL).astype(jnp.int32) * 4
    y6 = _k(idx, tpad)
    return y6.transpose(2, 4, 0, 1, 3).reshape(BATCH, HIST, DIM)


# speedup vs baseline: 1.8401x; 1.1041x over previous
import functools

import jax
import jax.numpy as jnp
from jax import lax
from jax.experimental import pallas as pl
from jax.experimental.pallas import tpu as pltpu
from jax.experimental.pallas import tpu_sc as plsc

VOCAB = 1000000
DIM = 32
BATCH = 16384
HIST = 50
TOTAL = BATCH * HIST

_info = plsc.get_sparse_core_info()
_NC = _info.num_cores
_NS = _info.num_subcores
_NW = _NC * _NS
_BW = BATCH // _NW
_BT = _BW // 128


def _make_kernel():
    mesh = plsc.VectorSubcoreMesh(core_axis_name="c", subcore_axis_name="s")

    @functools.partial(
        pl.kernel,
        mesh=mesh,
        out_type=jax.ShapeDtypeStruct((HIST, DIM // 8, BATCH // 128, 8, 128),
                                      jnp.float32),
        scratch_types=[
            pltpu.VMEM((HIST * _BW,), jnp.int32),
            [pltpu.VMEM((_BW, DIM), jnp.float32) for _ in range(2)],
            [pltpu.VMEM((DIM // 8, _BT, 8, 128), jnp.float32) for _ in range(2)],
            pltpu.SemaphoreType.DMA,
            [pltpu.SemaphoreType.DMA for _ in range(2)],
            [pltpu.SemaphoreType.DMA for _ in range(2)],
        ],
        compiler_params=pltpu.CompilerParams(
            use_tc_tiling_on_sc=False, needs_layout_passes=False
        ),
    )
    def k(idx_hbm, table_hbm, out_hbm, idx_all, rows, obuf, isem, gsems, ssems):
        w = lax.axis_index("s") * _NC + lax.axis_index("c")
        b0 = w * _BW

        idx_copies = []
        for h in range(HIST):
            idx_copies.append(pltpu.async_copy(
                idx_hbm.at[pl.ds(h * BATCH + b0, _BW)],
                idx_all.at[pl.ds(h * _BW, _BW)],
                isem,
            ))
        for c in idx_copies:
            c.wait()

        def start_gather(h, p):
            return pltpu.async_copy(
                table_hbm.at[idx_all.at[pl.ds(h * _BW, _BW)]],
                rows[p],
                gsems[p],
            )

        bvecs = [lax.iota(jnp.int32, 16) + 16 * j for j in range(_BW // 16)]

        def transpose_unit(p):
            rp, op = rows[p], obuf[p]

            def dtbody(dt, _):
                for di in range(8):
                    d = dt * 8 + di
                    dvec = jnp.full((16,), d, jnp.int32)
                    vs = [plsc.load_gather(rp, [bvecs[j], dvec])
                          for j in range(_BW // 16)]
                    for j in range(_BW // 16):
                        op[dt, j // 8, di, pl.ds((j % 8) * 16, 16)] = vs[j]
                return _

            lax.fori_loop(0, DIM // 8, dtbody, None)

        g0 = start_gather(0, 0)
        g1 = start_gather(1, 1)
        del g0, g1

        def store_ref(h):
            return out_hbm.at[h, :, pl.ds(_BT * w, _BT)]

        def unit(h, p):
            pltpu.make_async_copy(
                table_hbm.at[idx_all.at[pl.ds(h * _BW, _BW)]],
                rows[p],
                gsems[p],
            ).wait()

            @pl.when(h >= 2)
            def _():
                pltpu.make_async_copy(obuf[p], store_ref(h), ssems[p]).wait()

            transpose_unit(p)

            @pl.when(h + 2 < HIST)
            def _():
                start_gather(h + 2, p)

            pltpu.async_copy(obuf[p], store_ref(h), ssems[p])

        def base_body(base, _):
            unit(2 * base, 0)
            unit(2 * base + 1, 1)
            return _

        lax.fori_loop(0, HIST // 2, base_body, None)
        for p in range(2):
            pltpu.make_async_copy(obuf[p], store_ref(HIST - 2 + p), ssems[p]).wait()

    return k


_k = _make_kernel()


@jax.jit
def kernel(x, embeddings):
    tpad = jnp.pad(embeddings, ((0, 0), (0, 128 - DIM))).reshape(4 * VOCAB, DIM)
    idx = x.T.reshape(TOTAL).astype(jnp.int32) * 4
    y6 = _k(idx, tpad)
    return y6.transpose(2, 4, 0, 1, 3).reshape(BATCH, HIST, DIM)
